# baseline (speedup 1.0000x reference)
import functools

import jax
import jax.numpy as jnp
from jax import lax
from jax.experimental import pallas as pl
from jax.experimental.pallas import tpu as pltpu
from jax.experimental.pallas import tpu_sc as plsc

_L = 16


def _tc_transform(t_t, blk):
    n_tab = t_t.shape[1]
    grid = (n_tab + blk - 1) // blk

    def body(in_ref, p1_ref, p2_ref, v0_ref, v1_ref, v2_ref):
        r = in_ref[...]
        r3 = r[3, :]
        r4 = r[4, :]
        r5 = r[5, :]
        s = r3 * r3 + r4 * r4 + r5 * r5
        norm = jnp.sqrt(s)
        inv = 1.0 / jnp.maximum(norm, jnp.float32(1e-12))
        p1_ref[...] = 1000.0 * r[0, :]
        p2_ref[...] = 1000.0 * r[1, :]
        v0_ref[...] = -r5 * inv
        v1_ref[...] = r3 * inv
        v2_ref[...] = r4 * inv

    out_sds = jax.ShapeDtypeStruct((n_tab,), jnp.float32)
    return pl.pallas_call(
        body,
        grid=(grid,),
        in_specs=[pl.BlockSpec((6, blk), lambda i: (0, i))],
        out_specs=[pl.BlockSpec((blk,), lambda i: (i,))] * 5,
        out_shape=[out_sds] * 5,
    )(t_t)


def kernel(all_rays, indices, P_in, V_in):
    n_tab = all_rays.shape[0]
    B = indices.shape[0]
    n_pre = P_in.shape[0]
    info = plsc.get_sparse_core_info()
    NC, NS = info.num_cores, info.num_subcores
    NW = NC * NS
    mesh_kw = dict(core_axis_name="c", subcore_axis_name="s")

    comps = _tc_transform(all_rays.T, 32768)

    R = B // NW
    C = 8192
    n_chunks = R // C
    n_out = B + n_pre
    out_sds = jax.ShapeDtypeStruct((3, n_out), jnp.float32)

    @functools.partial(
        pl.kernel,
        out_type=(out_sds, out_sds),
        mesh=plsc.VectorSubcoreMesh(**mesh_kw),
        scratch_types=[
            pltpu.VMEM((C,), jnp.int32),
            pltpu.VMEM((5, C), jnp.float32),
            pltpu.VMEM((C,), jnp.float32),
            pltpu.SemaphoreType.DMA,
            pltpu.SemaphoreType.DMA,
            pltpu.SemaphoreType.DMA,
            pltpu.SemaphoreType.DMA,
            pltpu.SemaphoreType.DMA,
        ],
        compiler_params=pltpu.CompilerParams(
            needs_layout_passes=False, use_tc_tiling_on_sc=False),
    )
    def sample(p1, p2, v0, v1, v2, idx, p_in_t, v_in_t, p_out, v_out,
               idx_v, soa_v, zero_v, s0, s1, s2, s3, s4):
        wid = lax.axis_index("s") * NC + lax.axis_index("c")
        zvec = jnp.zeros((_L,), jnp.float32)

        def zinit(i, carry):
            zero_v[pl.ds(i * _L, _L)] = zvec
            return carry

        lax.fori_loop(0, C // _L, zinit, 0)

        @pl.when(wid == 0)
        def _():
            stage = soa_v.at[0, pl.ds(0, n_pre)]
            for j in range(3):
                pltpu.sync_copy(p_in_t.at[j], stage)
                pltpu.sync_copy(stage, p_out.at[j, pl.ds(0, n_pre)])
                pltpu.sync_copy(v_in_t.at[j], stage)
                pltpu.sync_copy(stage, v_out.at[j, pl.ds(0, n_pre)])

        def chunk_body(g, carry):
            base = wid * R + g * C
            pltpu.sync_copy(idx.at[pl.ds(base, C)], idx_v)
            cps = []
            for jj, (comp, sem) in enumerate(
                    ((p1, s0), (p2, s1), (v0, s2), (v1, s3), (v2, s4))):
                cps.append(pltpu.async_copy(comp.at[idx_v], soa_v.at[jj], sem))
            for cp in cps:
                cp.wait()
            dst = pl.ds(n_pre + base, C)
            pltpu.sync_copy(zero_v, p_out.at[0, dst])
            pltpu.sync_copy(soa_v.at[0], p_out.at[1, dst])
            pltpu.sync_copy(soa_v.at[1], p_out.at[2, dst])
            pltpu.sync_copy(soa_v.at[2], v_out.at[0, dst])
            pltpu.sync_copy(soa_v.at[3], v_out.at[1, dst])
            pltpu.sync_copy(soa_v.at[4], v_out.at[2, dst])
            return carry

        lax.fori_loop(0, n_chunks, chunk_body, 0)

    p_soa, v_soa = sample(*comps, indices.astype(jnp.int32), P_in.T, V_in.T)
    return (p_soa.T, v_soa.T)


# speedup vs baseline: 1.2145x; 1.2145x over previous
import functools

import jax
import jax.numpy as jnp
from jax import lax
from jax.experimental import pallas as pl
from jax.experimental.pallas import tpu as pltpu
from jax.experimental.pallas import tpu_sc as plsc

_L = 16


def _tc_transform(t_t, blk):
    n_tab = t_t.shape[1]
    grid = (n_tab + blk - 1) // blk

    def body(in_ref, p1_ref, p2_ref, v0_ref, v1_ref, v2_ref):
        r = in_ref[...]
        r3 = r[3, :]
        r4 = r[4, :]
        r5 = r[5, :]
        s = r3 * r3 + r4 * r4 + r5 * r5
        norm = jnp.sqrt(s)
        inv = 1.0 / jnp.maximum(norm, jnp.float32(1e-12))
        p1_ref[...] = 1000.0 * r[0, :]
        p2_ref[...] = 1000.0 * r[1, :]
        v0_ref[...] = -r5 * inv
        v1_ref[...] = r3 * inv
        v2_ref[...] = r4 * inv

    out_sds = jax.ShapeDtypeStruct((n_tab,), jnp.float32)
    return pl.pallas_call(
        body,
        grid=(grid,),
        in_specs=[pl.BlockSpec((6, blk), lambda i: (0, i))],
        out_specs=[pl.BlockSpec((blk,), lambda i: (i,))] * 5,
        out_shape=[out_sds] * 5,
    )(t_t)


def kernel(all_rays, indices, P_in, V_in):
    n_tab = all_rays.shape[0]
    B = indices.shape[0]
    n_pre = P_in.shape[0]
    info = plsc.get_sparse_core_info()
    NC, NS = info.num_cores, info.num_subcores
    NW = NC * NS
    mesh_kw = dict(core_axis_name="c", subcore_axis_name="s")

    comps = _tc_transform(all_rays.T, 32768)

    KC = 8000
    n_ck = n_tab // KC

    @functools.partial(
        pl.kernel,
        out_type=jax.ShapeDtypeStruct((n_tab * 8,), jnp.float32),
        mesh=plsc.VectorSubcoreMesh(**mesh_kw),
        scratch_types=[
            pltpu.VMEM((5, KC), jnp.float32),
            pltpu.VMEM((KC * 8,), jnp.float32),
        ],
        compiler_params=pltpu.CompilerParams(
            needs_layout_passes=False, use_tc_tiling_on_sc=False),
    )
    def interleave(p1, p2, v0, v1, v2, t8, in_v, out_v):
        wid = lax.axis_index("s") * NC + lax.axis_index("c")
        lane8 = lax.iota(jnp.int32, _L) * 8

        def chunk_body(ck, carry):
            c = wid + ck * NW

            @pl.when(c < n_ck)
            def _():
                start = c * KC
                for j, comp in enumerate((p1, p2, v0, v1, v2)):
                    pltpu.sync_copy(comp.at[pl.ds(start, KC)], in_v.at[j])

                def grp(i, carry2):
                    sl = pl.ds(i * _L, _L)
                    dst = lane8 + i * (_L * 8)
                    for j in range(5):
                        plsc.store_scatter(out_v, [dst + j + 1], in_v[j, sl])
                    return carry2

                lax.fori_loop(0, KC // _L, grp, 0)
                pltpu.sync_copy(out_v, t8.at[pl.ds(start * 8, KC * 8)])

            return carry

        lax.fori_loop(0, (n_ck + NW - 1) // NW, chunk_body, 0)

    t8 = interleave(*comps).reshape(n_tab, 8)

    R = B // NW
    C = 4096
    n_chunks = R // C
    n_out = B + n_pre
    out_sds = jax.ShapeDtypeStruct((3, n_out), jnp.float32)

    @functools.partial(
        pl.kernel,
        out_type=(out_sds, out_sds),
        mesh=plsc.VectorSubcoreMesh(**mesh_kw),
        scratch_types=[
            pltpu.VMEM((C,), jnp.int32),
            pltpu.VMEM((C,), jnp.int32),
            pltpu.VMEM((C, 8), jnp.float32),
            pltpu.VMEM((C, 8), jnp.float32),
            pltpu.VMEM((5, C), jnp.float32),
            pltpu.VMEM((C,), jnp.float32),
            pltpu.SemaphoreType.DMA,
            pltpu.SemaphoreType.DMA,
        ],
        compiler_params=pltpu.CompilerParams(
            needs_layout_passes=False, use_tc_tiling_on_sc=False),
    )
    def sample(table, idx, p_in_t, v_in_t, p_out, v_out,
               idx_v0, idx_v1, rows_v0, rows_v1, soa_v, zero_v, sem0, sem1):
        wid = lax.axis_index("s") * NC + lax.axis_index("c")
        zvec = jnp.zeros((_L,), jnp.float32)

        def zinit(i, carry):
            zero_v[pl.ds(i * _L, _L)] = zvec
            return carry

        lax.fori_loop(0, C // _L, zinit, 0)

        @pl.when(wid == 0)
        def _():
            stage = soa_v.at[0, pl.ds(0, n_pre)]
            for j in range(3):
                pltpu.sync_copy(p_in_t.at[j], stage)
                pltpu.sync_copy(stage, p_out.at[j, pl.ds(0, n_pre)])
                pltpu.sync_copy(v_in_t.at[j], stage)
                pltpu.sync_copy(stage, v_out.at[j, pl.ds(0, n_pre)])

        lane = lax.iota(jnp.int32, _L)
        idx_vs = (idx_v0, idx_v1)
        rows_vs = (rows_v0, rows_v1)
        sems = (sem0, sem1)

        def start(g, b):
            base = wid * R + g * C
            pltpu.sync_copy(idx.at[pl.ds(base, C)], idx_vs[b])
            return pltpu.async_copy(table.at[idx_vs[b]], rows_vs[b], sems[b])

        cp = start(0, 0)
        for g in range(n_chunks):
            b = g & 1
            cp_next = start(g + 1, 1 - b) if g + 1 < n_chunks else None
            cp.wait()
            rows_v = rows_vs[b]

            def grp(i, carry2, rows_v=rows_v):
                row = lane + i * _L
                sl = pl.ds(i * _L, _L)
                for jj in range(5):
                    soa_v[jj, sl] = plsc.load_gather(
                        rows_v, [row, jnp.full((_L,), jj + 1, jnp.int32)])
                return carry2

            lax.fori_loop(0, C // _L, grp, 0)
            base = wid * R + g * C
            dst = pl.ds(n_pre + base, C)
            pltpu.sync_copy(zero_v, p_out.at[0, dst])
            pltpu.sync_copy(soa_v.at[0], p_out.at[1, dst])
            pltpu.sync_copy(soa_v.at[1], p_out.at[2, dst])
            pltpu.sync_copy(soa_v.at[2], v_out.at[0, dst])
            pltpu.sync_copy(soa_v.at[3], v_out.at[1, dst])
            pltpu.sync_copy(soa_v.at[4], v_out.at[2, dst])
            cp = cp_next

    p_soa, v_soa = sample(t8, indices.astype(jnp.int32), P_in.T, V_in.T)
    return (p_soa.T, v_soa.T)


# speedup vs baseline: 1.3017x; 1.0718x over previous
import functools

import jax
import jax.numpy as jnp
from jax import lax
from jax.experimental import pallas as pl
from jax.experimental.pallas import tpu as pltpu
from jax.experimental.pallas import tpu_sc as plsc

_L = 16


def _tc_transform(t_t, blk):
    n_tab = t_t.shape[1]
    grid = (n_tab + blk - 1) // blk

    def body(in_ref, p1_ref, p2_ref, v0_ref, v1_ref, v2_ref):
        r = in_ref[...]
        r3 = r[3, :]
        r4 = r[4, :]
        r5 = r[5, :]
        s = r3 * r3 + r4 * r4 + r5 * r5
        norm = jnp.sqrt(s)
        inv = 1.0 / jnp.maximum(norm, jnp.float32(1e-12))
        p1_ref[...] = 1000.0 * r[0, :]
        p2_ref[...] = 1000.0 * r[1, :]
        v0_ref[...] = -r5 * inv
        v1_ref[...] = r3 * inv
        v2_ref[...] = r4 * inv

    out_sds = jax.ShapeDtypeStruct((n_tab,), jnp.float32)
    return pl.pallas_call(
        body,
        grid=(grid,),
        in_specs=[pl.BlockSpec((6, blk), lambda i: (0, i))],
        out_specs=[pl.BlockSpec((blk,), lambda i: (i,))] * 5,
        out_shape=[out_sds] * 5,
    )(t_t)


def kernel(all_rays, indices, P_in, V_in):
    n_tab = all_rays.shape[0]
    B = indices.shape[0]
    n_pre = P_in.shape[0]
    info = plsc.get_sparse_core_info()
    NC, NS = info.num_cores, info.num_subcores
    NW = NC * NS
    mesh_kw = dict(core_axis_name="c", subcore_axis_name="s")

    comps = _tc_transform(all_rays.T, 32768)

    KC = 4000
    n_ck = n_tab // KC
    rounds = (n_ck + NW - 1) // NW

    @functools.partial(
        pl.kernel,
        out_type=jax.ShapeDtypeStruct((n_tab * 8,), jnp.float32),
        mesh=plsc.VectorSubcoreMesh(**mesh_kw),
        scratch_types=[
            pltpu.VMEM((5, KC), jnp.float32),
            pltpu.VMEM((5, KC), jnp.float32),
            pltpu.VMEM((KC * 8,), jnp.float32),
            pltpu.SemaphoreType.DMA,
            pltpu.SemaphoreType.DMA,
        ],
        compiler_params=pltpu.CompilerParams(
            needs_layout_passes=False, use_tc_tiling_on_sc=False),
    )
    def interleave(p1, p2, v0, v1, v2, t8, in_v0, in_v1, out_v, sem0, sem1):
        wid = lax.axis_index("s") * NC + lax.axis_index("c")
        lane8 = lax.iota(jnp.int32, _L) * 8
        in_vs = (in_v0, in_v1)
        sems = (sem0, sem1)

        def start_in(ck, b):
            c = jnp.minimum(wid + ck * NW, n_ck - 1)
            st = c * KC
            cps = [
                pltpu.async_copy(comp.at[pl.ds(st, KC)], in_vs[b].at[j],
                                 sems[b])
                for j, comp in enumerate((p1, p2, v0, v1, v2))
            ]
            return cps, st

        cps, st = start_in(0, 0)
        for ck in range(rounds):
            b = ck & 1
            nxt = start_in(ck + 1, 1 - b) if ck + 1 < rounds else None
            for cp in cps:
                cp.wait()
            in_v = in_vs[b]

            def grp(i, carry2, in_v=in_v):
                sl = pl.ds(i * _L, _L)
                dst = lane8 + i * (_L * 8)
                for j in range(5):
                    plsc.store_scatter(out_v, [dst + j + 1], in_v[j, sl])
                return carry2

            lax.fori_loop(0, KC // _L, grp, 0)
            pltpu.sync_copy(out_v, t8.at[pl.ds(st * 8, KC * 8)])
            if nxt is not None:
                cps, st = nxt

    t8 = interleave(*comps).reshape(n_tab, 8)

    R = B // NW
    C = 4096
    n_chunks = R // C
    n_out = B + n_pre
    out_sds = jax.ShapeDtypeStruct((3, n_out), jnp.float32)

    @functools.partial(
        pl.kernel,
        out_type=(out_sds, out_sds),
        mesh=plsc.VectorSubcoreMesh(**mesh_kw),
        scratch_types=[
            pltpu.VMEM((C,), jnp.int32),
            pltpu.VMEM((C,), jnp.int32),
            pltpu.VMEM((C, 8), jnp.float32),
            pltpu.VMEM((C, 8), jnp.float32),
            pltpu.VMEM((5, C), jnp.float32),
            pltpu.VMEM((C,), jnp.float32),
            pltpu.SemaphoreType.DMA,
            pltpu.SemaphoreType.DMA,
        ],
        compiler_params=pltpu.CompilerParams(
            needs_layout_passes=False, use_tc_tiling_on_sc=False),
    )
    def sample(table, idx, p_in_t, v_in_t, p_out, v_out,
               idx_v0, idx_v1, rows_v0, rows_v1, soa_v, zero_v, sem0, sem1):
        wid = lax.axis_index("s") * NC + lax.axis_index("c")
        zvec = jnp.zeros((_L,), jnp.float32)

        def zinit(i, carry):
            zero_v[pl.ds(i * _L, _L)] = zvec
            return carry

        lax.fori_loop(0, C // _L, zinit, 0)

        @pl.when(wid == 0)
        def _():
            stage = soa_v.at[0, pl.ds(0, n_pre)]
            for j in range(3):
                pltpu.sync_copy(p_in_t.at[j], stage)
                pltpu.sync_copy(stage, p_out.at[j, pl.ds(0, n_pre)])
                pltpu.sync_copy(v_in_t.at[j], stage)
                pltpu.sync_copy(stage, v_out.at[j, pl.ds(0, n_pre)])

        lane = lax.iota(jnp.int32, _L)
        idx_vs = (idx_v0, idx_v1)
        rows_vs = (rows_v0, rows_v1)
        sems = (sem0, sem1)

        def start(g, b):
            base = wid * R + g * C
            pltpu.sync_copy(idx.at[pl.ds(base, C)], idx_vs[b])
            return pltpu.async_copy(table.at[idx_vs[b]], rows_vs[b], sems[b])

        cp = start(0, 0)
        for g in range(n_chunks):
            b = g & 1
            cp_next = start(g + 1, 1 - b) if g + 1 < n_chunks else None
            cp.wait()
            rows_v = rows_vs[b]

            def grp(i, carry2, rows_v=rows_v):
                row = lane + i * _L
                sl = pl.ds(i * _L, _L)
                for jj in range(5):
                    soa_v[jj, sl] = plsc.load_gather(
                        rows_v, [row, jnp.full((_L,), jj + 1, jnp.int32)])
                return carry2

            lax.fori_loop(0, C // _L, grp, 0)
            base = wid * R + g * C
            dst = pl.ds(n_pre + base, C)
            pltpu.sync_copy(zero_v, p_out.at[0, dst])
            pltpu.sync_copy(soa_v.at[0], p_out.at[1, dst])
            pltpu.sync_copy(soa_v.at[1], p_out.at[2, dst])
            pltpu.sync_copy(soa_v.at[2], v_out.at[0, dst])
            pltpu.sync_copy(soa_v.at[3], v_out.at[1, dst])
            pltpu.sync_copy(soa_v.at[4], v_out.at[2, dst])
            cp = cp_next

    p_soa, v_soa = sample(t8, indices.astype(jnp.int32), P_in.T, V_in.T)
    return (p_soa.T, v_soa.T)
